# baseline (speedup 1.0000x reference)
import math

import numpy as np

import jax
import jax.numpy as jnp
from jax import lax
from jax.experimental import pallas as pl
from jax.experimental.pallas import tpu as pltpu
from jax.experimental.pallas import tpu_sc as plsc

B = 1024
V = 100000
KS = (1, 5, 10, 20)

NC = 2
NS = 16
NW = NC * NS
L = 16

VSPLIT = 78848

RPW = B // NW
TROWS = RPW // 8
CW = 1408
CPR = 15
TAIL0 = VSPLIT + CPR * CW
TAILV = (V - TAIL0) // L
TOTAL = TROWS * CPR
NBUF = 4
JV = CW // L

TC_BC = 1024

NMET = 9

_mesh = plsc.VectorSubcoreMesh(core_axis_name="c", subcore_axis_name="s")
_sc_params = pltpu.CompilerParams(needs_layout_passes=False)


def _worker_id():
    return lax.axis_index("s") * NC + lax.axis_index("c")


def _pred_body(scores_hbm, labels_hbm, out_hbm, lab_ref, pblk_ref, pred_ref,
               gsem):
    wid = _worker_id()
    base_row = pl.multiple_of(wid * RPW, RPW)
    lane = lax.iota(jnp.int32, L)

    pltpu.sync_copy(labels_hbm.at[pl.ds(base_row, RPW)], lab_ref)
    for g in range(RPW // L):
        lab_v = lab_ref[pl.ds(g * L, L)]
        for r8 in range(L):
            r = g * L + r8
            lab = lab_v[r8]
            col0 = pl.multiple_of((lab // 128) * 128, 128)
            row0 = pl.multiple_of(base_row + (r // 8) * 8, 8)
            pltpu.async_copy(scores_hbm.at[pl.ds(row0, 8), pl.ds(col0, 128)],
                             pblk_ref.at[r], gsem)
    for r in range(RPW):
        pltpu.make_async_copy(scores_hbm.at[pl.ds(0, 8), pl.ds(0, 128)],
                              pblk_ref.at[0], gsem).wait()
    for g in range(RPW // L):
        lab_v = lab_ref[pl.ds(g * L, L)]
        r_v = g * L + lane
        s_v = lane % 8
        c_v = lab_v - (lab_v // 128) * 128
        pred_ref[pl.ds(g * L, L)] = plsc.load_gather(pblk_ref, [r_v, s_v, c_v])
    pltpu.sync_copy(pred_ref, out_hbm.at[pl.ds(base_row, RPW)])


_pred_call = pl.kernel(
    _pred_body,
    out_type=jax.ShapeDtypeStruct((B,), jnp.float32),
    mesh=_mesh,
    compiler_params=_sc_params,
    scratch_types=[
        pltpu.VMEM((RPW,), jnp.int32),
        pltpu.VMEM((RPW, 8, 128), jnp.float32),
        pltpu.VMEM((RPW,), jnp.float32),
        pltpu.SemaphoreType.DMA,
    ],
)


def _tc_count_body(pred_ref, scores_ref, out_ref):
    @pl.when(pl.program_id(0) == 0)
    def _():
        out_ref[...] = jnp.zeros_like(out_ref)
    x = scores_ref[...]
    p = pred_ref[...]
    out_ref[...] += jnp.sum((x > p).astype(jnp.float32), axis=1,
                            keepdims=True)


_tc_count = pl.pallas_call(
    _tc_count_body,
    grid=(VSPLIT // TC_BC,),
    in_specs=[
        pl.BlockSpec((B, 1), lambda j: (0, 0)),
        pl.BlockSpec((B, TC_BC), lambda j: (0, j)),
    ],
    out_specs=pl.BlockSpec((B, 1), lambda j: (0, 0)),
    out_shape=jax.ShapeDtypeStruct((B, 1), jnp.float32),
    compiler_params=pltpu.CompilerParams(
        dimension_semantics=("arbitrary",)),
)


def _scan_body(scores_hbm, pred_hbm, out_hbm,
               pred_ref, rank_ref, tl_ref,
               b0, b1, b2, b3, s0, s1, s2, s3, gsem):
    bufs = (b0, b1, b2, b3)
    sems = (s0, s1, s2, s3)
    cid = lax.axis_index("c")
    wid = _worker_id()
    base_row = pl.multiple_of(wid * RPW, RPW)
    lane = lax.iota(jnp.int32, L)

    pltpu.sync_copy(pred_hbm.at[pl.ds(base_row, RPW)], pred_ref)

    tail0 = pl.multiple_of((TAIL0 // 128 + cid * 0) * 128, 128)
    for tr in range(TROWS):
        row0 = pl.multiple_of(base_row + tr * 8, 8)
        pltpu.async_copy(scores_hbm.at[pl.ds(row0, 8), pl.ds(tail0, 128)],
                         tl_ref.at[tr], gsem)

    def issue_chunk(row0, c0, b):
        pltpu.async_copy(scores_hbm.at[pl.ds(row0, 8), pl.ds(c0, CW)],
                         bufs[b], sems[b])

    for b in range(NBUF):
        issue_chunk(pl.multiple_of(base_row, 8),
                    pl.multiple_of(VSPLIT + b * CW, 128), b)

    for tr in range(TROWS):
        pltpu.make_async_copy(scores_hbm.at[pl.ds(0, 8), pl.ds(0, 128)],
                              tl_ref.at[0], gsem).wait()

    zf = jnp.zeros((L,), jnp.float32)
    zi = jnp.zeros((L,), jnp.int32)

    def chunk_step(t_base, carry):
        cnts, ranks_v = carry
        cnts = list(cnts)
        for b in range(NBUF):
            t = t_base + b
            buf, sem = bufs[b], sems[b]
            pltpu.make_async_copy(scores_hbm.at[pl.ds(0, 8), pl.ds(0, CW)],
                                  buf, sem).wait()

            tr = t // CPR
            preds = [plsc.load_gather(pred_ref, [jnp.full((L,), tr * 8 + r,
                                                          jnp.int32)])
                     for r in range(8)]

            def inner(j, cv):
                out = []
                for r in range(8):
                    x = buf[r, pl.ds(j * L, L)]
                    out.append(cv[r]
                               + plsc.all_reduce_population_count(x > preds[r]))
                return tuple(out)

            cnts = list(lax.fori_loop(0, JV, inner, tuple(cnts), unroll=2))

            @pl.when(t + NBUF < TOTAL)
            def _():
                tn = t + NBUF
                trn = tn // CPR
                c0 = pl.multiple_of(VSPLIT + (tn % CPR) * CW, 128)
                row0 = pl.multiple_of(base_row + trn * 8, 8)
                issue_chunk(row0, c0, b)

            is_end = (t % CPR) == (CPR - 1)
            half = (tr % 2) * 8
            for r in range(8):
                sel = jnp.logical_and(is_end, lane == half + r)
                ranks_v = jnp.where(sel, cnts[r].astype(jnp.float32), ranks_v)
                cnts[r] = jnp.where(is_end, zi, cnts[r])

            g_end = jnp.logical_and(is_end, (tr % 2) == 1)

            @pl.when(g_end)
            def _():
                g0 = pl.multiple_of((tr // 2) * L, L)
                rank_ref[pl.ds(g0, L)] = ranks_v
        return tuple(cnts), ranks_v

    pl.loop(0, TOTAL, step=NBUF, init_carry=((zi,) * 8, zf))(chunk_step)

    for g in range(RPW // L):
        add_v = zf
        for h in range(2):
            tr = g * 2 + h
            for r in range(8):
                pred = plsc.load_gather(
                    pred_ref, [jnp.full((L,), tr * 8 + r, jnp.int32)])
                tc = zi
                for v in range(TAILV):
                    x = tl_ref[tr, r, pl.ds(v * L, L)]
                    tc = tc + plsc.all_reduce_population_count(x > pred)
                add_v = jnp.where(lane == h * 8 + r,
                                  tc.astype(jnp.float32), add_v)
        rank_ref[pl.ds(g * L, L)] = rank_ref[pl.ds(g * L, L)] + add_v

    pltpu.sync_copy(rank_ref, out_hbm.at[pl.ds(base_row, RPW)])


_scan_call = pl.kernel(
    _scan_body,
    out_type=jax.ShapeDtypeStruct((B,), jnp.float32),
    mesh=_mesh,
    compiler_params=_sc_params,
    scratch_types=[
        pltpu.VMEM((RPW,), jnp.float32),
        pltpu.VMEM((RPW,), jnp.float32),
        pltpu.VMEM((TROWS, 8, 128), jnp.float32),
        pltpu.VMEM((8, CW), jnp.float32),
        pltpu.VMEM((8, CW), jnp.float32),
        pltpu.VMEM((8, CW), jnp.float32),
        pltpu.VMEM((8, CW), jnp.float32),
        pltpu.SemaphoreType.DMA,
        pltpu.SemaphoreType.DMA,
        pltpu.SemaphoreType.DMA,
        pltpu.SemaphoreType.DMA,
        pltpu.SemaphoreType.DMA,
    ],
)


def _combine_body(sc_hbm, tc_hbm, table_hbm, out_hbm,
                  sbuf, tbuf, tab_ref, obuf, csem):
    wid = _worker_id()

    @pl.when(wid == 0)
    def _():
        pltpu.sync_copy(sc_hbm, sbuf)
        pltpu.sync_copy(tc_hbm, tbuf)
        pltpu.sync_copy(table_hbm, tab_ref)
        lane = lax.iota(jnp.int32, L)
        zf = jnp.zeros((L,), jnp.float32)

        def body(j, accs):
            r_v = sbuf[pl.ds(j * L, L)] + tbuf[pl.ds(j * L, L)]
            t_idx = jnp.minimum(r_v.astype(jnp.int32), 31)
            dcg_v = plsc.load_gather(tab_ref, [t_idx])
            out = []
            for k in KS:
                ind = (r_v < float(k)).astype(jnp.float32)
                out.append(accs[len(out)] + dcg_v * ind)
                out.append(accs[len(out)] + ind)
            out.append(accs[8] + 1.0 / (r_v + 1.0))
            return tuple(out)

        accs = lax.fori_loop(0, B // L, body, (zf,) * NMET)
        out_v = zf
        for i in range(NMET):
            s = jnp.sum(accs[i]) * (1.0 / B)
            out_v = jnp.where(lane == i, s, out_v)
        obuf[...] = out_v
        pltpu.sync_copy(obuf, out_hbm)


_combine_call = pl.kernel(
    _combine_body,
    out_type=jax.ShapeDtypeStruct((L,), jnp.float32),
    mesh=_mesh,
    compiler_params=_sc_params,
    scratch_types=[
        pltpu.VMEM((B,), jnp.float32),
        pltpu.VMEM((B,), jnp.float32),
        pltpu.VMEM((32,), jnp.float32),
        pltpu.VMEM((L,), jnp.float32),
        pltpu.SemaphoreType.DMA,
    ],
)

_TABLE = np.array([1.0 / math.log2(i + 2.0) for i in range(32)],
                  dtype=np.float32)


def kernel(scores, labels):
    predicts = _pred_call(scores, labels)
    tc_rank = _tc_count(predicts[:, None], scores)
    sc_rank = _scan_call(scores, predicts)
    out16 = _combine_call(sc_rank, tc_rank[:, 0], _TABLE)
    return out16[:NMET]


# speedup vs baseline: 1.0003x; 1.0003x over previous
import math

import numpy as np

import jax
import jax.numpy as jnp
from jax import lax
from jax.experimental import pallas as pl
from jax.experimental.pallas import tpu as pltpu
from jax.experimental.pallas import tpu_sc as plsc

B = 1024
V = 100000
KS = (1, 5, 10, 20)

NC = 2
NS = 16
NW = NC * NS
L = 16

VSPLIT = 67584

RPW = B // NW
TROWS = RPW // 8
CW = 2944
CPR = 11
TAIL0 = VSPLIT + CPR * CW
TAILV = (V - TAIL0) // L
TOTAL = TROWS * CPR
NBUF = 4
JV = CW // L

TC_BC = 1024
TC_GRID = VSPLIT // TC_BC

NMET = 9

_mesh = plsc.VectorSubcoreMesh(core_axis_name="c", subcore_axis_name="s")
_sc_params = pltpu.CompilerParams(needs_layout_passes=False)


def _worker_id():
    return lax.axis_index("s") * NC + lax.axis_index("c")


def _pred_body(scores_hbm, labels_hbm, out_hbm, lab_ref, pblk_ref, pred_ref,
               gsem):
    wid = _worker_id()
    base_row = pl.multiple_of(wid * RPW, RPW)
    lane = lax.iota(jnp.int32, L)

    pltpu.sync_copy(labels_hbm.at[pl.ds(base_row, RPW)], lab_ref)
    for g in range(RPW // L):
        lab_v = lab_ref[pl.ds(g * L, L)]
        for r8 in range(L):
            r = g * L + r8
            lab = lab_v[r8]
            col0 = pl.multiple_of((lab // 128) * 128, 128)
            row0 = pl.multiple_of(base_row + (r // 8) * 8, 8)
            pltpu.async_copy(scores_hbm.at[pl.ds(row0, 8), pl.ds(col0, 128)],
                             pblk_ref.at[r], gsem)
    for r in range(RPW):
        pltpu.make_async_copy(scores_hbm.at[pl.ds(0, 8), pl.ds(0, 128)],
                              pblk_ref.at[0], gsem).wait()
    for g in range(RPW // L):
        lab_v = lab_ref[pl.ds(g * L, L)]
        r_v = g * L + lane
        s_v = lane % 8
        c_v = lab_v - (lab_v // 128) * 128
        pred_ref[pl.ds(g * L, L)] = plsc.load_gather(pblk_ref, [r_v, s_v, c_v])
    pltpu.sync_copy(pred_ref, out_hbm.at[pl.ds(base_row, RPW)])


_pred_call = pl.kernel(
    _pred_body,
    out_type=jax.ShapeDtypeStruct((B,), jnp.float32),
    mesh=_mesh,
    compiler_params=_sc_params,
    scratch_types=[
        pltpu.VMEM((RPW,), jnp.int32),
        pltpu.VMEM((RPW, 8, 128), jnp.float32),
        pltpu.VMEM((RPW,), jnp.float32),
        pltpu.SemaphoreType.DMA,
    ],
)


def _tc_count_body(pred_ref, scores_ref, out_ref, acc_ref):
    j = pl.program_id(0)

    @pl.when(j == 0)
    def _():
        acc_ref[...] = jnp.zeros_like(acc_ref)

    x = scores_ref[...]
    p = pred_ref[...]
    a = acc_ref[...]
    for c in range(TC_BC // 128):
        a = a + (x[:, c * 128:(c + 1) * 128] > p).astype(jnp.float32)
    acc_ref[...] = a

    @pl.when(j == TC_GRID - 1)
    def _():
        out_ref[...] = jnp.sum(acc_ref[...], axis=1, keepdims=True)


_tc_count = pl.pallas_call(
    _tc_count_body,
    grid=(TC_GRID,),
    in_specs=[
        pl.BlockSpec((B, 1), lambda j: (0, 0)),
        pl.BlockSpec((B, TC_BC), lambda j: (0, j)),
    ],
    out_specs=pl.BlockSpec((B, 1), lambda j: (0, 0)),
    out_shape=jax.ShapeDtypeStruct((B, 1), jnp.float32),
    scratch_shapes=[pltpu.VMEM((B, 128), jnp.float32)],
    compiler_params=pltpu.CompilerParams(
        dimension_semantics=("arbitrary",)),
)


def _scan_body(scores_hbm, pred_hbm, out_hbm,
               pred_ref, rank_ref, tl_ref,
               b0, b1, b2, b3, s0, s1, s2, s3, gsem):
    bufs = (b0, b1, b2, b3)
    sems = (s0, s1, s2, s3)
    cid = lax.axis_index("c")
    wid = _worker_id()
    base_row = pl.multiple_of(wid * RPW, RPW)
    lane = lax.iota(jnp.int32, L)

    pltpu.sync_copy(pred_hbm.at[pl.ds(base_row, RPW)], pred_ref)

    tail0 = pl.multiple_of((TAIL0 // 128 + cid * 0) * 128, 128)
    for tr in range(TROWS):
        row0 = pl.multiple_of(base_row + tr * 8, 8)
        pltpu.async_copy(scores_hbm.at[pl.ds(row0, 8), pl.ds(tail0, 128)],
                         tl_ref.at[tr], gsem)

    def issue_chunk(row0, c0, b):
        pltpu.async_copy(scores_hbm.at[pl.ds(row0, 8), pl.ds(c0, CW)],
                         bufs[b], sems[b])

    for b in range(NBUF):
        issue_chunk(pl.multiple_of(base_row, 8),
                    pl.multiple_of(VSPLIT + b * CW, 128), b)

    for tr in range(TROWS):
        pltpu.make_async_copy(scores_hbm.at[pl.ds(0, 8), pl.ds(0, 128)],
                              tl_ref.at[0], gsem).wait()

    zf = jnp.zeros((L,), jnp.float32)
    zi = jnp.zeros((L,), jnp.int32)

    def chunk_step(t_base, carry):
        cnts, ranks_v = carry
        cnts = list(cnts)
        for b in range(NBUF):
            t = t_base + b
            buf, sem = bufs[b], sems[b]
            pltpu.make_async_copy(scores_hbm.at[pl.ds(0, 8), pl.ds(0, CW)],
                                  buf, sem).wait()

            tr = t // CPR
            preds = [plsc.load_gather(pred_ref, [jnp.full((L,), tr * 8 + r,
                                                          jnp.int32)])
                     for r in range(8)]

            def inner(j, cv):
                out = []
                for r in range(8):
                    x = buf[r, pl.ds(j * L, L)]
                    out.append(cv[r]
                               + plsc.all_reduce_population_count(x > preds[r]))
                return tuple(out)

            cnts = list(lax.fori_loop(0, JV, inner, tuple(cnts), unroll=2))

            @pl.when(t + NBUF < TOTAL)
            def _():
                tn = t + NBUF
                trn = tn // CPR
                c0 = pl.multiple_of(VSPLIT + (tn % CPR) * CW, 128)
                row0 = pl.multiple_of(base_row + trn * 8, 8)
                issue_chunk(row0, c0, b)

            is_end = (t % CPR) == (CPR - 1)
            half = (tr % 2) * 8
            for r in range(8):
                sel = jnp.logical_and(is_end, lane == half + r)
                ranks_v = jnp.where(sel, cnts[r].astype(jnp.float32), ranks_v)
                cnts[r] = jnp.where(is_end, zi, cnts[r])

            g_end = jnp.logical_and(is_end, (tr % 2) == 1)

            @pl.when(g_end)
            def _():
                g0 = pl.multiple_of((tr // 2) * L, L)
                rank_ref[pl.ds(g0, L)] = ranks_v
        return tuple(cnts), ranks_v

    pl.loop(0, TOTAL, step=NBUF, init_carry=((zi,) * 8, zf))(chunk_step)

    for g in range(RPW // L):
        add_v = zf
        for h in range(2):
            tr = g * 2 + h
            for r in range(8):
                pred = plsc.load_gather(
                    pred_ref, [jnp.full((L,), tr * 8 + r, jnp.int32)])
                tc = zi
                for v in range(TAILV):
                    x = tl_ref[tr, r, pl.ds(v * L, L)]
                    tc = tc + plsc.all_reduce_population_count(x > pred)
                add_v = jnp.where(lane == h * 8 + r,
                                  tc.astype(jnp.float32), add_v)
        rank_ref[pl.ds(g * L, L)] = rank_ref[pl.ds(g * L, L)] + add_v

    pltpu.sync_copy(rank_ref, out_hbm.at[pl.ds(base_row, RPW)])


_scan_call = pl.kernel(
    _scan_body,
    out_type=jax.ShapeDtypeStruct((B,), jnp.float32),
    mesh=_mesh,
    compiler_params=_sc_params,
    scratch_types=[
        pltpu.VMEM((RPW,), jnp.float32),
        pltpu.VMEM((RPW,), jnp.float32),
        pltpu.VMEM((TROWS, 8, 128), jnp.float32),
        pltpu.VMEM((8, CW), jnp.float32),
        pltpu.VMEM((8, CW), jnp.float32),
        pltpu.VMEM((8, CW), jnp.float32),
        pltpu.VMEM((8, CW), jnp.float32),
        pltpu.SemaphoreType.DMA,
        pltpu.SemaphoreType.DMA,
        pltpu.SemaphoreType.DMA,
        pltpu.SemaphoreType.DMA,
        pltpu.SemaphoreType.DMA,
    ],
)


def _combine_body(sc_hbm, tc_hbm, table_hbm, out_hbm,
                  sbuf, tbuf, tab_ref, obuf, csem):
    wid = _worker_id()

    @pl.when(wid == 0)
    def _():
        pltpu.sync_copy(sc_hbm, sbuf)
        pltpu.sync_copy(tc_hbm, tbuf)
        pltpu.sync_copy(table_hbm, tab_ref)
        lane = lax.iota(jnp.int32, L)
        zf = jnp.zeros((L,), jnp.float32)

        def body(j, accs):
            r_v = sbuf[pl.ds(j * L, L)] + tbuf[pl.ds(j * L, L)]
            t_idx = jnp.minimum(r_v.astype(jnp.int32), 31)
            dcg_v = plsc.load_gather(tab_ref, [t_idx])
            out = []
            for k in KS:
                ind = (r_v < float(k)).astype(jnp.float32)
                out.append(accs[len(out)] + dcg_v * ind)
                out.append(accs[len(out)] + ind)
            out.append(accs[8] + 1.0 / (r_v + 1.0))
            return tuple(out)

        accs = lax.fori_loop(0, B // L, body, (zf,) * NMET)
        out_v = zf
        for i in range(NMET):
            s = jnp.sum(accs[i]) * (1.0 / B)
            out_v = jnp.where(lane == i, s, out_v)
        obuf[...] = out_v
        pltpu.sync_copy(obuf, out_hbm)


_combine_call = pl.kernel(
    _combine_body,
    out_type=jax.ShapeDtypeStruct((L,), jnp.float32),
    mesh=_mesh,
    compiler_params=_sc_params,
    scratch_types=[
        pltpu.VMEM((B,), jnp.float32),
        pltpu.VMEM((B,), jnp.float32),
        pltpu.VMEM((32,), jnp.float32),
        pltpu.VMEM((L,), jnp.float32),
        pltpu.SemaphoreType.DMA,
    ],
)

_TABLE = np.array([1.0 / math.log2(i + 2.0) for i in range(32)],
                  dtype=np.float32)


def kernel(scores, labels):
    predicts = _pred_call(scores, labels)
    tc_rank = _tc_count(predicts[:, None], scores)
    sc_rank = _scan_call(scores, predicts)
    out16 = _combine_call(sc_rank, tc_rank[:, 0], _TABLE)
    return out16[:NMET]


# speedup vs baseline: 3.1596x; 3.1588x over previous
import numpy as np

import jax
import jax.numpy as jnp
from jax import lax
from jax.experimental import pallas as pl
from jax.experimental.pallas import tpu as pltpu
from jax.experimental.pallas import tpu_sc as plsc

B = 1024
V = 100000
KS = (1, 5, 10, 20)

NC = 2
NS = 16
NW = NC * NS
L = 16
LG = B // L

VSPLIT = 69280

IPW = (V - VSPLIT) // NW
CI = 24
NCHUNK = IPW // CI
NBUF = 4

TC_BR = 1024
TC_GRID = -(-VSPLIT // TC_BR)
TC_LAST_SLABS = (VSPLIT - (TC_GRID - 1) * TC_BR) // 8

NMET = 9

_mesh = plsc.VectorSubcoreMesh(core_axis_name="c", subcore_axis_name="s")
_sc_params = pltpu.CompilerParams(needs_layout_passes=False)


def _worker_id():
    return lax.axis_index("s") * NC + lax.axis_index("c")


PWORK = B // 128
PPH = 2
PWIN = 128 // PPH


def _pred_body(scores_hbm, labels_hbm, out_hbm, lab_ref, pblk_ref, stage_ref,
               gsem):
    wid = _worker_id()
    lane = lax.iota(jnp.int32, L)

    @pl.when(wid < PWORK)
    def _():
        base_b = pl.multiple_of(wid * 128, 128)
        pltpu.sync_copy(labels_hbm.at[pl.ds(base_b, 128)], lab_ref)
        for ph in range(PPH):
            for g in range(PWIN // L):
                lab_v = lab_ref[pl.ds(ph * PWIN + g * L, L)]
                for r8 in range(L):
                    lab = lab_v[r8]
                    row0 = pl.multiple_of((lab // 8) * 8, 8)
                    pltpu.async_copy(
                        scores_hbm.at[pl.ds(row0, 8), pl.ds(base_b, 128)],
                        pblk_ref.at[g * L + r8], gsem)
            for k in range(PWIN):
                pltpu.make_async_copy(
                    scores_hbm.at[pl.ds(0, 8), pl.ds(0, 128)],
                    pblk_ref.at[0], gsem).wait()
            for g in range(PWIN // L):
                lab_v = lab_ref[pl.ds(ph * PWIN + g * L, L)]
                k_v = g * L + lane
                srow_v = lab_v - (lab_v // 8) * 8
                scol_v = ph * PWIN + g * L + lane
                p_v = plsc.load_gather(pblk_ref, [k_v, srow_v, scol_v])
                for r in range(8):
                    stage_ref[r, pl.ds(ph * PWIN + g * L, L)] = p_v
        pltpu.sync_copy(stage_ref,
                        out_hbm.at[pl.ds(0, 8), pl.ds(base_b, 128)])


_pred_call = pl.kernel(
    _pred_body,
    out_type=jax.ShapeDtypeStruct((8, B), jnp.float32),
    mesh=_mesh,
    compiler_params=_sc_params,
    scratch_types=[
        pltpu.VMEM((128,), jnp.int32),
        pltpu.VMEM((PWIN, 8, 128), jnp.float32),
        pltpu.VMEM((8, 128), jnp.float32),
        pltpu.SemaphoreType.DMA,
    ],
)


def _tc_count_body(pred_ref, scores_ref, out_ref, acc_ref):
    j = pl.program_id(0)

    @pl.when(j == 0)
    def _():
        acc_ref[...] = jnp.zeros_like(acc_ref)

    p = pred_ref[...]

    def accumulate(n_slabs):
        x = scores_ref[...]
        a = acc_ref[...]
        for s in range(n_slabs):
            a = a + (x[s * 8:(s + 1) * 8, :] > p).astype(jnp.float32)
        acc_ref[...] = a

    @pl.when(j < TC_GRID - 1)
    def _():
        accumulate(TC_BR // 8)

    @pl.when(j == TC_GRID - 1)
    def _():
        accumulate(TC_LAST_SLABS)

    @pl.when(j == TC_GRID - 1)
    def _():
        out_ref[...] = acc_ref[...]


_tc_count = pl.pallas_call(
    _tc_count_body,
    grid=(TC_GRID,),
    in_specs=[
        pl.BlockSpec((8, B), lambda j: (0, 0)),
        pl.BlockSpec((TC_BR, B), lambda j: (j, 0)),
    ],
    out_specs=pl.BlockSpec((8, B), lambda j: (0, 0)),
    out_shape=jax.ShapeDtypeStruct((8, B), jnp.float32),
    scratch_shapes=[pltpu.VMEM((8, B), jnp.float32)],
    compiler_params=pltpu.CompilerParams(
        dimension_semantics=("arbitrary",)),
)


def _scan_body(scores_hbm, pred_hbm, out_hbm,
               pred_ref, acc_ref, stage_ref,
               b0, b1, b2, b3, s0, s1, s2, s3, gsem):
    bufs = (b0, b1, b2, b3)
    sems = (s0, s1, s2, s3)
    wid = _worker_id()
    item0 = pl.multiple_of(VSPLIT + wid * IPW, 8)

    pltpu.sync_copy(pred_hbm, pred_ref)

    def issue_chunk(t, b):
        src = pl.multiple_of(item0 + t * CI, 8)
        pltpu.async_copy(scores_hbm.at[pl.ds(src, CI)], bufs[b], sems[b])

    for b in range(NBUF):
        issue_chunk(b, b)

    zf = jnp.zeros((L,), jnp.float32)

    @pl.loop(0, LG)
    def _(lg):
        acc_ref[pl.ds(lg * L, L)] = zf

    @pl.loop(0, NCHUNK, step=NBUF)
    def _(t_base):
        for b in range(NBUF):
            t = t_base + b
            buf, sem = bufs[b], sems[b]
            pltpu.make_async_copy(scores_hbm.at[pl.ds(0, CI)],
                                  buf, sem).wait()

            def body(lg, _):
                o = lg * L
                pred_v = pred_ref[0, pl.ds(o, L)]
                acc_v = acc_ref[pl.ds(o, L)]
                for it in range(CI):
                    x = buf[it, pl.ds(o, L)]
                    acc_v = acc_v + (x > pred_v).astype(jnp.float32)
                acc_ref[pl.ds(o, L)] = acc_v
                return 0

            lax.fori_loop(0, LG, body, 0)

            @pl.when(t + NBUF < NCHUNK)
            def _():
                issue_chunk(t + NBUF, b)

    @pl.loop(0, LG)
    def _(lg):
        o = lg * L
        stage_ref[0, pl.ds(o, L)] = acc_ref[pl.ds(o, L)]
        for r in range(1, 8):
            stage_ref[r, pl.ds(o, L)] = zf

    row_out = pl.multiple_of(wid * 8, 8)
    pltpu.sync_copy(stage_ref, out_hbm.at[pl.ds(row_out, 8)])


_scan_call = pl.kernel(
    _scan_body,
    out_type=jax.ShapeDtypeStruct((NW * 8, B), jnp.float32),
    mesh=_mesh,
    compiler_params=_sc_params,
    scratch_types=[
        pltpu.VMEM((8, B), jnp.float32),
        pltpu.VMEM((B,), jnp.float32),
        pltpu.VMEM((8, B), jnp.float32),
        pltpu.VMEM((CI, B), jnp.float32),
        pltpu.VMEM((CI, B), jnp.float32),
        pltpu.VMEM((CI, B), jnp.float32),
        pltpu.VMEM((CI, B), jnp.float32),
        pltpu.SemaphoreType.DMA,
        pltpu.SemaphoreType.DMA,
        pltpu.SemaphoreType.DMA,
        pltpu.SemaphoreType.DMA,
        pltpu.SemaphoreType.DMA,
    ],
)


def _combine_body(tc_ref, sc_ref, out_ref):
    rank = (jnp.sum(tc_ref[...], axis=0, keepdims=True)
            + jnp.sum(sc_ref[...], axis=0, keepdims=True))
    inv_b = 1.0 / B
    dcg = np.float32(np.log(2.0)) / jnp.log(rank + 2.0)
    res = []
    for k in KS:
        ind = (rank < float(k)).astype(jnp.float32)
        res.append(jnp.sum(dcg * ind) * inv_b)
        res.append(jnp.sum(ind) * inv_b)
    res.append(jnp.sum(1.0 / (rank + 1.0)) * inv_b)
    out_ref[...] = jnp.stack(res)


_tc_combine = pl.pallas_call(
    _combine_body,
    in_specs=[
        pl.BlockSpec((8, B), lambda: (0, 0)),
        pl.BlockSpec((NW * 8, B), lambda: (0, 0)),
    ],
    out_specs=pl.BlockSpec((NMET,), lambda: (0,)),
    out_shape=jax.ShapeDtypeStruct((NMET,), jnp.float32),
)


def kernel(scores, labels):
    scores_t = scores.T
    pred8 = _pred_call(scores_t, labels)
    tc8 = _tc_count(pred8, scores_t)
    sc256 = _scan_call(scores_t, pred8)
    return _tc_combine(tc8, sc256)


# speedup vs baseline: 3.4020x; 1.0767x over previous
import numpy as np

import jax
import jax.numpy as jnp
from jax import lax
from jax.experimental import pallas as pl
from jax.experimental.pallas import tpu as pltpu
from jax.experimental.pallas import tpu_sc as plsc

B = 1024
V = 100000
KS = (1, 5, 10, 20)

NC = 2
NS = 16
NW = NC * NS
L = 16
LG = B // L

VSPLIT = 63136

IPW = (V - VSPLIT) // NW
CI = 24
NCHUNK = IPW // CI
NBUF = 4

TC_BR = 2048
TC_GRID = -(-VSPLIT // TC_BR)
TC_LAST_SLABS = (VSPLIT - (TC_GRID - 1) * TC_BR) // 8

NMET = 9

_mesh = plsc.VectorSubcoreMesh(core_axis_name="c", subcore_axis_name="s")
_sc_params = pltpu.CompilerParams(needs_layout_passes=False)


def _worker_id():
    return lax.axis_index("s") * NC + lax.axis_index("c")


PWORK = B // 128
PPH = 2
PWIN = 128 // PPH


def _pred_body(scores_hbm, labels_hbm, out_hbm, lab_ref, pblk_ref, stage_ref,
               gsem):
    wid = _worker_id()
    lane = lax.iota(jnp.int32, L)

    @pl.when(wid < PWORK)
    def _():
        base_b = pl.multiple_of(wid * 128, 128)
        pltpu.sync_copy(labels_hbm.at[pl.ds(base_b, 128)], lab_ref)
        for ph in range(PPH):
            for g in range(PWIN // L):
                lab_v = lab_ref[pl.ds(ph * PWIN + g * L, L)]
                for r8 in range(L):
                    lab = lab_v[r8]
                    row0 = pl.multiple_of((lab // 8) * 8, 8)
                    pltpu.async_copy(
                        scores_hbm.at[pl.ds(row0, 8), pl.ds(base_b, 128)],
                        pblk_ref.at[g * L + r8], gsem)
            for k in range(PWIN):
                pltpu.make_async_copy(
                    scores_hbm.at[pl.ds(0, 8), pl.ds(0, 128)],
                    pblk_ref.at[0], gsem).wait()
            for g in range(PWIN // L):
                lab_v = lab_ref[pl.ds(ph * PWIN + g * L, L)]
                k_v = g * L + lane
                srow_v = lab_v - (lab_v // 8) * 8
                scol_v = ph * PWIN + g * L + lane
                p_v = plsc.load_gather(pblk_ref, [k_v, srow_v, scol_v])
                for r in range(8):
                    stage_ref[r, pl.ds(ph * PWIN + g * L, L)] = p_v
        pltpu.sync_copy(stage_ref,
                        out_hbm.at[pl.ds(0, 8), pl.ds(base_b, 128)])


_pred_call = pl.kernel(
    _pred_body,
    out_type=jax.ShapeDtypeStruct((8, B), jnp.float32),
    mesh=_mesh,
    compiler_params=_sc_params,
    scratch_types=[
        pltpu.VMEM((128,), jnp.int32),
        pltpu.VMEM((PWIN, 8, 128), jnp.float32),
        pltpu.VMEM((8, 128), jnp.float32),
        pltpu.SemaphoreType.DMA,
    ],
)


def _tc_count_body(pred_ref, scores_ref, out_ref, acc_ref):
    j = pl.program_id(0)

    @pl.when(j == 0)
    def _():
        acc_ref[...] = jnp.zeros_like(acc_ref)

    p = pred_ref[...]

    def accumulate(n_slabs):
        x = scores_ref[...]
        a = acc_ref[...]
        for s in range(n_slabs):
            a = a + (x[s * 8:(s + 1) * 8, :] > p).astype(jnp.float32)
        acc_ref[...] = a

    @pl.when(j < TC_GRID - 1)
    def _():
        accumulate(TC_BR // 8)

    @pl.when(j == TC_GRID - 1)
    def _():
        accumulate(TC_LAST_SLABS)

    @pl.when(j == TC_GRID - 1)
    def _():
        out_ref[...] = acc_ref[...]


_tc_count = pl.pallas_call(
    _tc_count_body,
    grid=(TC_GRID,),
    in_specs=[
        pl.BlockSpec((8, B), lambda j: (0, 0)),
        pl.BlockSpec((TC_BR, B), lambda j: (j, 0)),
    ],
    out_specs=pl.BlockSpec((8, B), lambda j: (0, 0)),
    out_shape=jax.ShapeDtypeStruct((8, B), jnp.float32),
    scratch_shapes=[pltpu.VMEM((8, B), jnp.float32)],
    compiler_params=pltpu.CompilerParams(
        dimension_semantics=("arbitrary",)),
)


def _scan_body(scores_hbm, pred_hbm, out_hbm,
               pred_ref, acc_ref, stage_ref,
               b0, b1, b2, b3, s0, s1, s2, s3, gsem):
    bufs = (b0, b1, b2, b3)
    sems = (s0, s1, s2, s3)
    wid = _worker_id()
    item0 = pl.multiple_of(VSPLIT + wid * IPW, 8)

    pltpu.sync_copy(pred_hbm, pred_ref)

    def issue_chunk(t, b):
        src = pl.multiple_of(item0 + t * CI, 8)
        pltpu.async_copy(scores_hbm.at[pl.ds(src, CI)], bufs[b], sems[b])

    for b in range(NBUF):
        issue_chunk(b, b)

    zf = jnp.zeros((L,), jnp.float32)

    @pl.loop(0, LG)
    def _(lg):
        acc_ref[pl.ds(lg * L, L)] = zf

    @pl.loop(0, NCHUNK, step=NBUF)
    def _(t_base):
        for b in range(NBUF):
            t = t_base + b
            buf, sem = bufs[b], sems[b]
            pltpu.make_async_copy(scores_hbm.at[pl.ds(0, CI)],
                                  buf, sem).wait()

            def body(lg, _):
                o = lg * L
                pred_v = pred_ref[0, pl.ds(o, L)]
                acc_v = acc_ref[pl.ds(o, L)]
                for it in range(CI):
                    x = buf[it, pl.ds(o, L)]
                    acc_v = acc_v + (x > pred_v).astype(jnp.float32)
                acc_ref[pl.ds(o, L)] = acc_v
                return 0

            lax.fori_loop(0, LG, body, 0)

            @pl.when(t + NBUF < NCHUNK)
            def _():
                issue_chunk(t + NBUF, b)

    @pl.loop(0, LG)
    def _(lg):
        o = lg * L
        stage_ref[0, pl.ds(o, L)] = acc_ref[pl.ds(o, L)]
        for r in range(1, 8):
            stage_ref[r, pl.ds(o, L)] = zf

    row_out = pl.multiple_of(wid * 8, 8)
    pltpu.sync_copy(stage_ref, out_hbm.at[pl.ds(row_out, 8)])


_scan_call = pl.kernel(
    _scan_body,
    out_type=jax.ShapeDtypeStruct((NW * 8, B), jnp.float32),
    mesh=_mesh,
    compiler_params=_sc_params,
    scratch_types=[
        pltpu.VMEM((8, B), jnp.float32),
        pltpu.VMEM((B,), jnp.float32),
        pltpu.VMEM((8, B), jnp.float32),
        pltpu.VMEM((CI, B), jnp.float32),
        pltpu.VMEM((CI, B), jnp.float32),
        pltpu.VMEM((CI, B), jnp.float32),
        pltpu.VMEM((CI, B), jnp.float32),
        pltpu.SemaphoreType.DMA,
        pltpu.SemaphoreType.DMA,
        pltpu.SemaphoreType.DMA,
        pltpu.SemaphoreType.DMA,
        pltpu.SemaphoreType.DMA,
    ],
)


def _combine_body(tc_ref, sc_ref, out_ref):
    rank = (jnp.sum(tc_ref[...], axis=0, keepdims=True)
            + jnp.sum(sc_ref[...], axis=0, keepdims=True))
    inv_b = 1.0 / B
    dcg = np.float32(np.log(2.0)) / jnp.log(rank + 2.0)
    res = []
    for k in KS:
        ind = (rank < float(k)).astype(jnp.float32)
        res.append(jnp.sum(dcg * ind) * inv_b)
        res.append(jnp.sum(ind) * inv_b)
    res.append(jnp.sum(1.0 / (rank + 1.0)) * inv_b)
    out_ref[...] = jnp.stack(res)


_tc_combine = pl.pallas_call(
    _combine_body,
    in_specs=[
        pl.BlockSpec((8, B), lambda: (0, 0)),
        pl.BlockSpec((NW * 8, B), lambda: (0, 0)),
    ],
    out_specs=pl.BlockSpec((NMET,), lambda: (0,)),
    out_shape=jax.ShapeDtypeStruct((NMET,), jnp.float32),
)


def kernel(scores, labels):
    scores_t = scores.T
    pred8 = _pred_call(scores_t, labels)
    tc8 = _tc_count(pred8, scores_t)
    sc256 = _scan_call(scores_t, pred8)
    return _tc_combine(tc8, sc256)


# speedup vs baseline: 3.4560x; 1.0159x over previous
import numpy as np

import jax
import jax.numpy as jnp
from jax import lax
from jax.experimental import pallas as pl
from jax.experimental.pallas import tpu as pltpu
from jax.experimental.pallas import tpu_sc as plsc

B = 1024
V = 100000
KS = (1, 5, 10, 20)

NC = 2
NS = 16
NW = NC * NS
L = 16
LG = B // L

VSPLIT = 63136

IPW = (V - VSPLIT) // NW
CI = 24
NCHUNK = IPW // CI
NBUF = 4

TC_BR = 4096
TC_GRID = -(-VSPLIT // TC_BR)
TC_LAST_SLABS = (VSPLIT - (TC_GRID - 1) * TC_BR) // 8

NMET = 9

_mesh = plsc.VectorSubcoreMesh(core_axis_name="c", subcore_axis_name="s")
_sc_params = pltpu.CompilerParams(needs_layout_passes=False)


def _worker_id():
    return lax.axis_index("s") * NC + lax.axis_index("c")


PQW = 32


def _pred_body(scores_hbm, labels_hbm, o0, o1, o2, o3,
               lab_ref, pblk_ref, stage_ref, gsem):
    wid = _worker_id()
    lane = lax.iota(jnp.int32, L)
    q = wid % 4
    base_b = pl.multiple_of((wid // 4) * 128, 128)
    base_l = pl.multiple_of(wid * PQW, PQW)

    pltpu.sync_copy(labels_hbm.at[pl.ds(base_l, PQW)], lab_ref)
    for g in range(PQW // L):
        lab_v = lab_ref[pl.ds(g * L, L)]
        for r8 in range(L):
            lab = lab_v[r8]
            row0 = pl.multiple_of((lab // 8) * 8, 8)
            pltpu.async_copy(
                scores_hbm.at[pl.ds(row0, 8), pl.ds(base_b, 128)],
                pblk_ref.at[g * L + r8], gsem)
    zf = jnp.zeros((L,), jnp.float32)
    for r in range(8):
        for c in range(128 // L):
            stage_ref[r, pl.ds(c * L, L)] = zf
    for k in range(PQW):
        pltpu.make_async_copy(scores_hbm.at[pl.ds(0, 8), pl.ds(0, 128)],
                              pblk_ref.at[0], gsem).wait()
    for g in range(PQW // L):
        lab_v = lab_ref[pl.ds(g * L, L)]
        k_v = g * L + lane
        srow_v = lab_v - (lab_v // 8) * 8
        scol_v = q * PQW + g * L + lane
        p_v = plsc.load_gather(pblk_ref, [k_v, srow_v, scol_v])
        o = pl.multiple_of(q * PQW + g * L, L)
        for r in range(8):
            stage_ref[r, pl.ds(o, L)] = p_v
    outs = (o0, o1, o2, o3)
    for qq in range(4):
        @pl.when(q == qq)
        def _(qq=qq):
            pltpu.sync_copy(stage_ref,
                            outs[qq].at[pl.ds(0, 8), pl.ds(base_b, 128)])


_pred_call = pl.kernel(
    _pred_body,
    out_type=[jax.ShapeDtypeStruct((8, B), jnp.float32)] * 4,
    mesh=_mesh,
    compiler_params=_sc_params,
    scratch_types=[
        pltpu.VMEM((PQW,), jnp.int32),
        pltpu.VMEM((PQW, 8, 128), jnp.float32),
        pltpu.VMEM((8, 128), jnp.float32),
        pltpu.SemaphoreType.DMA,
    ],
)


def _tc_count_body(p0_ref, p1_ref, p2_ref, p3_ref, scores_ref, out_ref,
                   acc_ref):
    j = pl.program_id(0)

    @pl.when(j == 0)
    def _():
        acc_ref[...] = jnp.zeros_like(acc_ref)

    p = (p0_ref[...] + p1_ref[...]) + (p2_ref[...] + p3_ref[...])

    def accumulate(n_slabs):
        x = scores_ref[...]
        a = acc_ref[...]
        for s in range(n_slabs):
            a = a + (x[s * 8:(s + 1) * 8, :] > p).astype(jnp.float32)
        acc_ref[...] = a

    @pl.when(j < TC_GRID - 1)
    def _():
        accumulate(TC_BR // 8)

    @pl.when(j == TC_GRID - 1)
    def _():
        accumulate(TC_LAST_SLABS)

    @pl.when(j == TC_GRID - 1)
    def _():
        out_ref[...] = acc_ref[...]


_tc_count = pl.pallas_call(
    _tc_count_body,
    grid=(TC_GRID,),
    in_specs=[
        pl.BlockSpec((8, B), lambda j: (0, 0)),
        pl.BlockSpec((8, B), lambda j: (0, 0)),
        pl.BlockSpec((8, B), lambda j: (0, 0)),
        pl.BlockSpec((8, B), lambda j: (0, 0)),
        pl.BlockSpec((TC_BR, B), lambda j: (j, 0)),
    ],
    out_specs=pl.BlockSpec((8, B), lambda j: (0, 0)),
    out_shape=jax.ShapeDtypeStruct((8, B), jnp.float32),
    scratch_shapes=[pltpu.VMEM((8, B), jnp.float32)],
    compiler_params=pltpu.CompilerParams(
        dimension_semantics=("arbitrary",)),
)


def _scan_body(scores_hbm, p0_hbm, p1_hbm, p2_hbm, p3_hbm, out_hbm,
               pred_ref, acc_ref, stage_ref,
               b0, b1, b2, b3, s0, s1, s2, s3, gsem):
    bufs = (b0, b1, b2, b3)
    sems = (s0, s1, s2, s3)
    wid = _worker_id()
    item0 = pl.multiple_of(VSPLIT + wid * IPW, 8)

    pltpu.sync_copy(p0_hbm, b0.at[pl.ds(0, 8)])
    pltpu.sync_copy(p1_hbm, b0.at[pl.ds(8, 8)])
    pltpu.sync_copy(p2_hbm, b0.at[pl.ds(16, 8)])
    pltpu.sync_copy(p3_hbm, b1.at[pl.ds(0, 8)])

    @pl.loop(0, LG)
    def _(lg):
        o = lg * L
        pred_ref[pl.ds(o, L)] = ((b0[0, pl.ds(o, L)] + b0[8, pl.ds(o, L)])
                                 + (b0[16, pl.ds(o, L)] + b1[0, pl.ds(o, L)]))

    def issue_chunk(t, b):
        src = pl.multiple_of(item0 + t * CI, 8)
        pltpu.async_copy(scores_hbm.at[pl.ds(src, CI)], bufs[b], sems[b])

    for b in range(NBUF):
        issue_chunk(b, b)

    zf = jnp.zeros((L,), jnp.float32)

    @pl.loop(0, LG)
    def _(lg):
        acc_ref[pl.ds(lg * L, L)] = zf

    @pl.loop(0, NCHUNK, step=NBUF)
    def _(t_base):
        for b in range(NBUF):
            t = t_base + b
            buf, sem = bufs[b], sems[b]
            pltpu.make_async_copy(scores_hbm.at[pl.ds(0, CI)],
                                  buf, sem).wait()

            def body(lg, _):
                o = lg * L
                pred_v = pred_ref[pl.ds(o, L)]
                acc_v = acc_ref[pl.ds(o, L)]
                for it in range(CI):
                    x = buf[it, pl.ds(o, L)]
                    acc_v = acc_v + (x > pred_v).astype(jnp.float32)
                acc_ref[pl.ds(o, L)] = acc_v
                return 0

            lax.fori_loop(0, LG, body, 0)

            @pl.when(t + NBUF < NCHUNK)
            def _():
                issue_chunk(t + NBUF, b)

    @pl.loop(0, LG)
    def _(lg):
        o = lg * L
        stage_ref[0, pl.ds(o, L)] = acc_ref[pl.ds(o, L)]
        for r in range(1, 8):
            stage_ref[r, pl.ds(o, L)] = zf

    row_out = pl.multiple_of(wid * 8, 8)
    pltpu.sync_copy(stage_ref, out_hbm.at[pl.ds(row_out, 8)])


_scan_call = pl.kernel(
    _scan_body,
    out_type=jax.ShapeDtypeStruct((NW * 8, B), jnp.float32),
    mesh=_mesh,
    compiler_params=_sc_params,
    scratch_types=[
        pltpu.VMEM((B,), jnp.float32),
        pltpu.VMEM((B,), jnp.float32),
        pltpu.VMEM((8, B), jnp.float32),
        pltpu.VMEM((CI, B), jnp.float32),
        pltpu.VMEM((CI, B), jnp.float32),
        pltpu.VMEM((CI, B), jnp.float32),
        pltpu.VMEM((CI, B), jnp.float32),
        pltpu.SemaphoreType.DMA,
        pltpu.SemaphoreType.DMA,
        pltpu.SemaphoreType.DMA,
        pltpu.SemaphoreType.DMA,
        pltpu.SemaphoreType.DMA,
    ],
)


def _combine_body(tc_ref, sc_ref, out_ref):
    rank = (jnp.sum(tc_ref[...], axis=0, keepdims=True)
            + jnp.sum(sc_ref[...], axis=0, keepdims=True))
    inv_b = 1.0 / B
    dcg = np.float32(np.log(2.0)) / jnp.log(rank + 2.0)
    res = []
    for k in KS:
        ind = (rank < float(k)).astype(jnp.float32)
        res.append(jnp.sum(dcg * ind) * inv_b)
        res.append(jnp.sum(ind) * inv_b)
    res.append(jnp.sum(1.0 / (rank + 1.0)) * inv_b)
    out_ref[...] = jnp.stack(res)


_tc_combine = pl.pallas_call(
    _combine_body,
    in_specs=[
        pl.BlockSpec((8, B), lambda: (0, 0)),
        pl.BlockSpec((NW * 8, B), lambda: (0, 0)),
    ],
    out_specs=pl.BlockSpec((NMET,), lambda: (0,)),
    out_shape=jax.ShapeDtypeStruct((NMET,), jnp.float32),
)


def kernel(scores, labels):
    scores_t = scores.T
    p0, p1, p2, p3 = _pred_call(scores_t, labels)
    tc8 = _tc_count(p0, p1, p2, p3, scores_t)
    sc256 = _scan_call(scores_t, p0, p1, p2, p3)
    return _tc_combine(tc8, sc256)
